# baseline (speedup 1.0000x reference)
import functools

import jax
import jax.numpy as jnp
from jax import lax
from jax.experimental import pallas as pl
from jax.experimental.pallas import tpu as pltpu
from jax.experimental.pallas import tpu_sc as plsc

B = 8
C = 256
NUM_ANCHORS = 9
DISC = 28
N_SPARSE = 2048
LEVEL_HW = [(64, 64), (32, 32), (16, 16), (8, 8), (4, 4)]
HWS = [h * w for (h, w) in LEVEL_HW]

NUM_WORKERS = 32
TPW = N_SPARSE // NUM_WORKERS


N_PAD = N_SPARSE + NUM_WORKERS * 5


CH = 16
NCH = TPW // CH


def _gather_body(f0, f1, f2, f3, f4, sb_hbm, sl_hbm, so_hbm, out_hbm,
                 *scr):
    feats = [f0, f1, f2, f3, f4]
    sb_v, sl_v, so_v, sidx_v, didx_v = scr[0:5]
    gc = [list(scr[5 + 4 * l:5 + 4 * l + 4]) for l in range(5)]
    sem_g = [list(scr[25 + 4 * l:25 + 4 * l + 4]) for l in range(5)]
    sem_s = scr[45]
    wid = lax.axis_index("s") * 2 + lax.axis_index("c")
    base = wid * TPW
    pltpu.sync_copy(sb_hbm.at[pl.ds(base, TPW)], sb_v)
    pltpu.sync_copy(sl_hbm.at[pl.ds(base, TPW)], sl_v)
    pltpu.sync_copy(so_hbm.at[pl.ds(base, TPW)], so_v)
    lanes = lax.iota(jnp.int32, 16)
    for lvl in range(5):
        for q in range(NCH):
            sidx_v[lvl, pl.ds(q * 16, 16)] = lanes
            didx_v[lvl, pl.ds(q * 16, 16)] = (
                lanes * 0 + N_SPARSE + wid * 5 + lvl)
    cnt = [jnp.int32(0)] * 5
    for q in range(NCH):
        sl16 = pl.ds(q * 16, 16)
        bv = sb_v[sl16]
        lv = sl_v[sl16]
        ov = so_v[sl16]
        hw = jnp.where(
            lv == 0, HWS[0],
            jnp.where(lv == 1, HWS[1],
                      jnp.where(lv == 2, HWS[2],
                                jnp.where(lv == 3, HWS[3], HWS[4]))))
        row = bv * hw + jnp.bitwise_and(ov, hw - 1)
        dst = base + q * 16 + lanes
        for lvl in range(5):
            m = lv == lvl
            mi = m.astype(jnp.int32)
            p = plsc.cumsum(mi) - 1 + cnt[lvl]
            l16 = lanes * 0 + lvl
            plsc.store_scatter(sidx_v, [l16, p], row, mask=m)
            plsc.store_scatter(didx_v, [l16, p], dst, mask=m)
            cnt[lvl] = cnt[lvl] + jnp.sum(mi)
    copies = {}
    for lvl in range(5):
        for k in range(NCH):
            @pl.when(cnt[lvl] > k * CH)
            def _(lvl=lvl, k=k):
                pltpu.async_copy(
                    feats[lvl].at[sidx_v.at[lvl, pl.ds(k * CH, CH)]],
                    gc[lvl][k], sem_g[lvl][k])
    for lvl in range(5):
        for k in range(NCH):
            @pl.when(cnt[lvl] > k * CH)
            def _(lvl=lvl, k=k):
                pltpu.make_async_copy(
                    feats[lvl].at[sidx_v.at[lvl, pl.ds(k * CH, CH)]],
                    gc[lvl][k], sem_g[lvl][k]).wait()
                pltpu.async_copy(
                    gc[lvl][k],
                    out_hbm.at[didx_v.at[lvl, pl.ds(k * CH, CH)]], sem_s)
    for lvl in range(5):
        for k in range(NCH):
            @pl.when(cnt[lvl] > k * CH)
            def _(lvl=lvl, k=k):
                pltpu.make_async_copy(
                    gc[lvl][k],
                    out_hbm.at[didx_v.at[lvl, pl.ds(k * CH, CH)]],
                    sem_s).wait()


def _gather(f0, f1, f2, f3, f4, sb, sl, so):
    mesh = plsc.VectorSubcoreMesh(core_axis_name="c", subcore_axis_name="s")
    return pl.kernel(
        _gather_body,
        out_type=jax.ShapeDtypeStruct((N_PAD, C), jnp.float32),
        mesh=mesh,
        compiler_params=pltpu.CompilerParams(needs_layout_passes=False),
        scratch_types=(
            [pltpu.VMEM((TPW,), jnp.int32)] * 3
            + [pltpu.VMEM((5, TPW), jnp.int32)] * 2
            + [pltpu.VMEM((CH, C), jnp.float32)] * 20
            + [pltpu.SemaphoreType.DMA] * 21
        ),
    )(f0, f1, f2, f3, f4, sb, sl, so)


TOK_TILE = 1024
D2 = DISC * DISC


def _head_body(anchr_ref, g_ref, w1_ref, b1_ref, wpt_ref, bp_ref, out_ref):
    xt = lax.dot_general(w1_ref[...], g_ref[...], (((0,), (1,)), ((), ())),
                         preferred_element_type=jnp.float32)
    xt = jnp.maximum(xt + b1_ref[:, 0:1], 0.0)
    arow = anchr_ref[0:1, :]
    oht = (lax.broadcasted_iota(jnp.int32, (NUM_ANCHORS, 1), 0)
           == arow).astype(jnp.float32)
    acc = lax.dot_general(bp_ref[...], oht, (((0,), (0,)), ((), ())),
                          preferred_element_type=jnp.float32)
    for a in range(NUM_ANCHORS):
        xm = jnp.where(arow == a, xt, 0.0)
        acc = acc + lax.dot_general(
            wpt_ref[a], xm, (((1,), (0,)), ((), ())),
            preferred_element_type=jnp.float32)
    out_ref[...] = acc


def _head(anchors_row, g, W1, b1_col, WpT, bp):
    grid = (N_SPARSE // TOK_TILE,)
    return pl.pallas_call(
        _head_body,
        grid=grid,
        in_specs=[
            pl.BlockSpec((8, TOK_TILE), lambda i: (0, i)),
            pl.BlockSpec((TOK_TILE, C), lambda i: (i, 0)),
            pl.BlockSpec((C, C), lambda i: (0, 0)),
            pl.BlockSpec((C, 8), lambda i: (0, 0)),
            pl.BlockSpec((NUM_ANCHORS, D2, C), lambda i: (0, 0, 0)),
            pl.BlockSpec((NUM_ANCHORS, D2), lambda i: (0, 0)),
        ],
        out_specs=pl.BlockSpec((D2, TOK_TILE), lambda i: (0, i)),
        out_shape=jax.ShapeDtypeStruct((D2, N_SPARSE), jnp.float32),
    )(anchors_row, g, W1, b1_col, WpT, bp)


def kernel(feat0, feat1, feat2, feat3, feat4,
           sparse_batch, sparse_layers, sparse_off, sparse_anchor_idx,
           W1, b1, Wp, bp):
    f = [jnp.transpose(x, (0, 2, 3, 1)).reshape(-1, C)
         for x in (feat0, feat1, feat2, feat3, feat4)]
    g = _gather(f[0], f[1], f[2], f[3], f[4],
                sparse_batch, sparse_layers, sparse_off)
    out_t = _head(jnp.tile(sparse_anchor_idx.reshape(1, N_SPARSE), (8, 1)),
                  g, W1, jnp.tile(b1.reshape(C, 1), (1, 8)),
                  jnp.transpose(Wp, (0, 2, 1)), bp)
    return out_t.T.reshape(N_SPARSE, DISC, DISC)


# speedup vs baseline: 1.0045x; 1.0045x over previous
import jax
import jax.numpy as jnp
from jax import lax
from jax.experimental import pallas as pl
from jax.experimental.pallas import tpu as pltpu
from jax.experimental.pallas import tpu_sc as plsc

B = 8
C = 256
NUM_ANCHORS = 9
DISC = 28
N_SPARSE = 2048
LEVEL_HW = [(64, 64), (32, 32), (16, 16), (8, 8), (4, 4)]
HWS = [h * w for (h, w) in LEVEL_HW]

NUM_WORKERS = 32
TPW = N_SPARSE // NUM_WORKERS


N_PAD = N_SPARSE + NUM_WORKERS * 5


CH = 16
NCH = TPW // CH


def _gather_body(f0, f1, f2, f3, f4, sb_hbm, sl_hbm, so_hbm, out_hbm,
                 *scr):
    feats = [f0, f1, f2, f3, f4]
    sb_v, sl_v, so_v, sidx_v, didx_v = scr[0:5]
    gc = [list(scr[5 + 4 * l:5 + 4 * l + 4]) for l in range(5)]
    sem_g = [list(scr[25 + 4 * l:25 + 4 * l + 4]) for l in range(5)]
    sem_s = scr[45]
    wid = lax.axis_index("s") * 2 + lax.axis_index("c")
    base = wid * TPW
    pltpu.sync_copy(sb_hbm.at[pl.ds(base, TPW)], sb_v)
    pltpu.sync_copy(sl_hbm.at[pl.ds(base, TPW)], sl_v)
    pltpu.sync_copy(so_hbm.at[pl.ds(base, TPW)], so_v)
    lanes = lax.iota(jnp.int32, 16)
    for lvl in range(5):
        for q in range(NCH):
            sidx_v[lvl, pl.ds(q * 16, 16)] = lanes
            didx_v[lvl, pl.ds(q * 16, 16)] = (
                lanes * 0 + N_SPARSE + wid * 5 + lvl)
    cnt = [jnp.int32(0)] * 5
    for q in range(NCH):
        sl16 = pl.ds(q * 16, 16)
        bv = sb_v[sl16]
        lv = sl_v[sl16]
        ov = so_v[sl16]
        hw = jnp.where(
            lv == 0, HWS[0],
            jnp.where(lv == 1, HWS[1],
                      jnp.where(lv == 2, HWS[2],
                                jnp.where(lv == 3, HWS[3], HWS[4]))))
        row = bv * hw + jnp.bitwise_and(ov, hw - 1)
        dst = base + q * 16 + lanes
        for lvl in range(5):
            m = lv == lvl
            mi = m.astype(jnp.int32)
            p = plsc.cumsum(mi) - 1 + cnt[lvl]
            l16 = lanes * 0 + lvl
            plsc.store_scatter(sidx_v, [l16, p], row, mask=m)
            plsc.store_scatter(didx_v, [l16, p], dst, mask=m)
            cnt[lvl] = cnt[lvl] + jnp.sum(mi)
    for lvl in range(5):
        for k in range(NCH):
            @pl.when(cnt[lvl] > k * CH)
            def _(lvl=lvl, k=k):
                pltpu.async_copy(
                    feats[lvl].at[sidx_v.at[lvl, pl.ds(k * CH, CH)]],
                    gc[lvl][k], sem_g[lvl][k])
    for lvl in range(5):
        for k in range(NCH):
            @pl.when(cnt[lvl] > k * CH)
            def _(lvl=lvl, k=k):
                pltpu.make_async_copy(
                    feats[lvl].at[sidx_v.at[lvl, pl.ds(k * CH, CH)]],
                    gc[lvl][k], sem_g[lvl][k]).wait()
                pltpu.async_copy(
                    gc[lvl][k],
                    out_hbm.at[didx_v.at[lvl, pl.ds(k * CH, CH)]], sem_s)
    for lvl in range(5):
        for k in range(NCH):
            @pl.when(cnt[lvl] > k * CH)
            def _(lvl=lvl, k=k):
                pltpu.make_async_copy(
                    gc[lvl][k],
                    out_hbm.at[didx_v.at[lvl, pl.ds(k * CH, CH)]],
                    sem_s).wait()


def _gather(f0, f1, f2, f3, f4, sb, sl, so):
    mesh = plsc.VectorSubcoreMesh(core_axis_name="c", subcore_axis_name="s")
    return pl.kernel(
        _gather_body,
        out_type=jax.ShapeDtypeStruct((N_PAD, C), jnp.float32),
        mesh=mesh,
        compiler_params=pltpu.CompilerParams(needs_layout_passes=False),
        scratch_types=(
            [pltpu.VMEM((TPW,), jnp.int32)] * 3
            + [pltpu.VMEM((5, TPW), jnp.int32)] * 2
            + [pltpu.VMEM((CH, C), jnp.float32)] * 20
            + [pltpu.SemaphoreType.DMA] * 21
        ),
    )(f0, f1, f2, f3, f4, sb, sl, so)


TOK_TILE = 1024
D2 = DISC * DISC


def _head_body(anchr_ref, g_ref, w1_ref, b1_ref, wpt_ref, bp_ref, out_ref):
    xt = lax.dot_general(w1_ref[...], g_ref[...], (((0,), (1,)), ((), ())),
                         preferred_element_type=jnp.float32)
    xt = jnp.maximum(xt + b1_ref[:, 0:1], 0.0)
    arow = anchr_ref[0:1, :]
    oht = (lax.broadcasted_iota(jnp.int32, (NUM_ANCHORS, 1), 0)
           == arow).astype(jnp.float32)
    acc = lax.dot_general(bp_ref[...], oht, (((0,), (0,)), ((), ())),
                          preferred_element_type=jnp.float32)
    for a in range(NUM_ANCHORS):
        xm = jnp.where(arow == a, xt, 0.0)
        acc = acc + lax.dot_general(
            wpt_ref[a], xm, (((1,), (0,)), ((), ())),
            preferred_element_type=jnp.float32)
    out_ref[...] = acc


def _head(anchors_row, g, W1, b1_col, WpT, bp):
    grid = (N_SPARSE // TOK_TILE,)
    return pl.pallas_call(
        _head_body,
        grid=grid,
        in_specs=[
            pl.BlockSpec((8, TOK_TILE), lambda i: (0, i)),
            pl.BlockSpec((TOK_TILE, C), lambda i: (i, 0)),
            pl.BlockSpec((C, C), lambda i: (0, 0)),
            pl.BlockSpec((C, 8), lambda i: (0, 0)),
            pl.BlockSpec((NUM_ANCHORS, D2, C), lambda i: (0, 0, 0)),
            pl.BlockSpec((NUM_ANCHORS, D2), lambda i: (0, 0)),
        ],
        out_specs=pl.BlockSpec((D2, TOK_TILE), lambda i: (0, i)),
        out_shape=jax.ShapeDtypeStruct((D2, N_SPARSE), jnp.float32),
    )(anchors_row, g, W1, b1_col, WpT, bp)


def kernel(feat0, feat1, feat2, feat3, feat4,
           sparse_batch, sparse_layers, sparse_off, sparse_anchor_idx,
           W1, b1, Wp, bp):
    f = [jnp.transpose(x, (0, 2, 3, 1)).reshape(-1, C)
         for x in (feat0, feat1, feat2, feat3, feat4)]
    g = _gather(f[0], f[1], f[2], f[3], f[4],
                sparse_batch, sparse_layers, sparse_off)
    out_t = _head(jnp.tile(sparse_anchor_idx.reshape(1, N_SPARSE), (8, 1)),
                  g, W1, jnp.tile(b1.reshape(C, 1), (1, 8)),
                  jnp.transpose(Wp, (0, 2, 1)), bp)
    return out_t.T.reshape(N_SPARSE, DISC, DISC)
